# baseline (speedup 1.0000x reference)
import jax
import jax.numpy as jnp
from jax import lax
from jax.experimental import pallas as pl
from jax.experimental.pallas import tpu as pltpu
from jax.experimental.pallas import tpu_sc as plsc

N = 10000
NNZ = 320000
DIN = 128
NC = 2
NS = 16
LANES = 16
FH = DIN // NC

CH = 80
PER_TILE = NNZ // NS
NCHUNK = PER_TILE // CH

DUMP = 3 * N
NTOT = 3 * N + 1
ZPT = (3 * N) // NS
WPT = (3 * N) // NS


def _sc_body(values_hbm, row_hbm, col_hbm, out_hbm,
             vals_v, rc_v, idx_v, acc_sh, sem_in0, sem_in1, sem_sc):
    c = lax.axis_index("c")
    s = lax.axis_index("s")

    zeros = jnp.zeros((LANES,), jnp.float32)

    def zrow(i, carry):
        for q in range(FH // LANES):
            vals_v[0, i, pl.ds(q * LANES, LANES)] = zeros
        return carry

    lax.fori_loop(0, CH, zrow, 0)
    zsrc = vals_v.at[0]
    for z in range(ZPT // CH):
        pltpu.sync_copy(zsrc, acc_sh.at[pl.ds(s * ZPT + z * CH, CH)])
    if ZPT % CH:
        pltpu.sync_copy(zsrc.at[pl.ds(0, ZPT % CH)],
                        acc_sh.at[pl.ds(s * ZPT + (ZPT // CH) * CH, ZPT % CH)])

    @pl.when(s == 0)
    def _():
        pltpu.sync_copy(zsrc.at[pl.ds(0, 1)], acc_sh.at[pl.ds(DUMP, 1)])

    plsc.subcore_barrier()

    sems = (sem_in0, sem_in1)

    def issue_inputs(k, b):
        e0 = pl.multiple_of(s * PER_TILE + k * CH, CH)
        sem = sems[b]
        pltpu.async_copy(values_hbm.at[pl.ds(e0, CH), pl.ds(c * FH, FH)],
                         vals_v.at[b], sem)
        pltpu.async_copy(row_hbm.at[pl.ds(e0, CH)], rc_v.at[b, 0], sem)
        pltpu.async_copy(col_hbm.at[pl.ds(e0, CH)], rc_v.at[b, 1], sem)

    def wait_inputs(b):
        sem = sems[b]
        pltpu.make_async_copy(values_hbm.at[pl.ds(0, CH), pl.ds(0, FH)],
                              vals_v.at[b], sem).wait()
        pltpu.make_async_copy(row_hbm.at[pl.ds(0, CH)], rc_v.at[b, 0],
                              sem).wait()
        pltpu.make_async_copy(col_hbm.at[pl.ds(0, CH)], rc_v.at[b, 1],
                              sem).wait()

    def drain_scatter(n):
        for _ in range(n):
            pltpu.make_async_copy(vals_v.at[0],
                                  acc_sh.at[idx_v.at[0, 0]], sem_sc).wait()

    issue_inputs(0, 0)

    def chunk(k, b, first, last_chunk, carry, guard_issue=False):
        if not first:
            drain_scatter(2)

            @pl.when(carry)
            def _():
                drain_scatter(1)

        if not last_chunk:
            if guard_issue:
                @pl.when(k + 1 < NCHUNK)
                def _():
                    issue_inputs(k + 1, 1 - b)
            else:
                issue_inputs(k + 1, 1 - b)

        wait_inputs(b)
        nd = jnp.zeros((LANES,), jnp.float32)
        for q in range(CH // LANES):
            o = q * LANES
            r = rc_v[b, 0, pl.ds(o, LANES)]
            cc = rc_v[b, 1, pl.ds(o, LANES)]
            m = r == cc
            nd = nd + jnp.where(m, 1.0, 0.0)
            idx_v[b, 0, pl.ds(o, LANES)] = r
            idx_v[b, 1, pl.ds(o, LANES)] = cc + N
            idx_v[b, 2, pl.ds(o, LANES)] = jnp.where(m, r + 2 * N, DUMP)
        tot = nd[0]
        for i in range(1, LANES):
            tot = tot + nd[i]
        ndiag = tot > 0.5

        vb = vals_v.at[b]
        pltpu.async_copy(vb, acc_sh.at[idx_v.at[b, 0]], sem_sc, add=True)
        pltpu.async_copy(vb, acc_sh.at[idx_v.at[b, 1]], sem_sc, add=True)

        @pl.when(ndiag)
        def _():
            pltpu.async_copy(vb, acc_sh.at[idx_v.at[b, 2]], sem_sc, add=True)

        return ndiag

    def chunk_pair(k2, carry):
        k = k2 * 2
        carry = chunk(k, 0, False, False, carry)
        carry = chunk(k + 1, 1, False, False, carry, guard_issue=True)
        return carry

    carry = chunk(0, 0, True, False, jnp.bool_(False))
    carry = chunk(1, 1, False, False, carry)
    carry = lax.fori_loop(1, NCHUNK // 2, chunk_pair, carry)
    last = carry
    drain_scatter(2)

    @pl.when(last)
    def _():
        drain_scatter(1)

    plsc.subcore_barrier()

    r0 = s * WPT
    pltpu.sync_copy(acc_sh.at[pl.ds(r0, WPT)],
                    out_hbm.at[pl.ds(r0, WPT), pl.ds(c * FH, FH)])


_sc_call = pl.kernel(
    _sc_body,
    out_type=jax.ShapeDtypeStruct((3 * N, DIN), jnp.float32),
    mesh=plsc.VectorSubcoreMesh(core_axis_name="c", subcore_axis_name="s",
                                num_cores=NC, num_subcores=NS),
    compiler_params=pltpu.CompilerParams(use_tc_tiling_on_sc=False),
    scratch_types=[
        pltpu.VMEM((2, CH, FH), jnp.float32),
        pltpu.VMEM((2, 2, CH), jnp.int32),
        pltpu.VMEM((2, 3, CH), jnp.int32),
        pltpu.VMEM_SHARED((NTOT, FH), jnp.float32),
        pltpu.SemaphoreType.DMA,
        pltpu.SemaphoreType.DMA,
        pltpu.SemaphoreType.DMA,
    ],
)


def _tc_body(acc_ref, w_ref, b_ref, y_ref):
    f32 = jnp.float32
    a_row = acc_ref[0:N, :]
    a_col = acc_ref[N:2 * N, :]
    a_diag = acc_ref[2 * N:3 * N, :]
    ones = jnp.ones((8, N), f32)
    s_all = jnp.dot(ones, a_row, preferred_element_type=f32,
                    precision=jax.lax.Precision.HIGHEST)[0:1, :]
    s_diag = jnp.dot(ones, a_diag, preferred_element_type=f32,
                     precision=jax.lax.Precision.HIGHEST)[0:1, :]
    const = (jnp.dot(s_diag, w_ref[1], preferred_element_type=f32)
             + jnp.dot(s_all, w_ref[4], preferred_element_type=f32)
             + b_ref[0, 0])
    y = (jnp.dot(a_diag, w_ref[0], preferred_element_type=f32)
         + jnp.dot(a_row, w_ref[2], preferred_element_type=f32)
         + jnp.dot(a_col, w_ref[3], preferred_element_type=f32))
    y_ref[...] = y + const


_tc_call = pl.pallas_call(
    _tc_body,
    out_shape=jax.ShapeDtypeStruct((N, DIN), jnp.float32),
    in_specs=[
        pl.BlockSpec(memory_space=pltpu.VMEM),
        pl.BlockSpec(memory_space=pltpu.VMEM),
        pl.BlockSpec(memory_space=pltpu.SMEM),
    ],
    out_specs=pl.BlockSpec(memory_space=pltpu.VMEM),
)


@jax.jit
def kernel(values, row, col, weights, bias):
    row = row.astype(jnp.int32)
    col = col.astype(jnp.int32)
    acc = _sc_call(values, row, col)
    return _tc_call(acc, weights, bias.reshape(1, 1).astype(jnp.float32))


# speedup vs baseline: 1.0177x; 1.0177x over previous
import jax
import jax.numpy as jnp
from jax import lax
from jax.experimental import pallas as pl
from jax.experimental.pallas import tpu as pltpu
from jax.experimental.pallas import tpu_sc as plsc

N = 10000
NNZ = 320000
DIN = 128
NC = 2
NS = 16
LANES = 16
FH = DIN // NC

CH = 80
PER_TILE = NNZ // NS
NCHUNK = PER_TILE // CH

DUMP = 3 * N
NTOT = 3 * N + 1
ZPT = (3 * N) // NS
WPT = (3 * N) // NS


def _sc_body(values_hbm, row_hbm, col_hbm, out_hbm,
             vals_v, rc_v, idx_v, acc_sh, sem_in0, sem_in1, sem_sc):
    c = lax.axis_index("c")
    s = lax.axis_index("s")

    sems = (sem_in0, sem_in1)

    def issue_inputs(k, b):
        e0 = pl.multiple_of(s * PER_TILE + k * CH, CH)
        sem = sems[b]
        pltpu.async_copy(values_hbm.at[pl.ds(e0, CH), pl.ds(c * FH, FH)],
                         vals_v.at[b], sem)
        pltpu.async_copy(row_hbm.at[pl.ds(e0, CH)], rc_v.at[b, 0], sem)
        pltpu.async_copy(col_hbm.at[pl.ds(e0, CH)], rc_v.at[b, 1], sem)

    def wait_inputs(b):
        sem = sems[b]
        pltpu.make_async_copy(values_hbm.at[pl.ds(0, CH), pl.ds(0, FH)],
                              vals_v.at[b], sem).wait()
        pltpu.make_async_copy(row_hbm.at[pl.ds(0, CH)], rc_v.at[b, 0],
                              sem).wait()
        pltpu.make_async_copy(col_hbm.at[pl.ds(0, CH)], rc_v.at[b, 1],
                              sem).wait()

    def drain_scatter(n):
        for _ in range(n):
            pltpu.make_async_copy(vals_v.at[0],
                                  acc_sh.at[idx_v.at[0, 0]], sem_sc).wait()

    issue_inputs(0, 0)

    zeros = jnp.zeros((LANES,), jnp.float32)

    def zrow(i, carry):
        for q in range(FH // LANES):
            vals_v[1, i, pl.ds(q * LANES, LANES)] = zeros
        return carry

    lax.fori_loop(0, CH, zrow, 0)
    zsrc = vals_v.at[1]
    for z in range(ZPT // CH):
        pltpu.sync_copy(zsrc, acc_sh.at[pl.ds(s * ZPT + z * CH, CH)])
    if ZPT % CH:
        pltpu.sync_copy(zsrc.at[pl.ds(0, ZPT % CH)],
                        acc_sh.at[pl.ds(s * ZPT + (ZPT // CH) * CH, ZPT % CH)])

    @pl.when(s == 0)
    def _():
        pltpu.sync_copy(zsrc.at[pl.ds(0, 1)], acc_sh.at[pl.ds(DUMP, 1)])

    plsc.subcore_barrier()

    def chunk(k, b, first, last_chunk, carry, guard_issue=False):
        if not first:
            drain_scatter(2)

            @pl.when(carry)
            def _():
                drain_scatter(1)

        if not last_chunk:
            if guard_issue:
                @pl.when(k + 1 < NCHUNK)
                def _():
                    issue_inputs(k + 1, 1 - b)
            else:
                issue_inputs(k + 1, 1 - b)

        wait_inputs(b)
        nd = jnp.zeros((LANES,), jnp.float32)
        for q in range(CH // LANES):
            o = q * LANES
            r = rc_v[b, 0, pl.ds(o, LANES)]
            cc = rc_v[b, 1, pl.ds(o, LANES)]
            m = r == cc
            nd = nd + jnp.where(m, 1.0, 0.0)
            idx_v[b, 0, pl.ds(o, LANES)] = r
            idx_v[b, 1, pl.ds(o, LANES)] = cc + N
            idx_v[b, 2, pl.ds(o, LANES)] = jnp.where(m, r + 2 * N, DUMP)
        tot = nd[0]
        for i in range(1, LANES):
            tot = tot + nd[i]
        ndiag = tot > 0.5

        vb = vals_v.at[b]
        pltpu.async_copy(vb, acc_sh.at[idx_v.at[b, 0]], sem_sc, add=True)
        pltpu.async_copy(vb, acc_sh.at[idx_v.at[b, 1]], sem_sc, add=True)

        @pl.when(ndiag)
        def _():
            pltpu.async_copy(vb, acc_sh.at[idx_v.at[b, 2]], sem_sc, add=True)

        return ndiag

    def chunk_pair(k2, carry):
        k = k2 * 2
        carry = chunk(k, 0, False, False, carry)
        carry = chunk(k + 1, 1, False, False, carry, guard_issue=True)
        return carry

    carry = chunk(0, 0, True, False, jnp.bool_(False))
    carry = chunk(1, 1, False, False, carry)
    carry = lax.fori_loop(1, NCHUNK // 2, chunk_pair, carry)
    last = carry
    drain_scatter(2)

    @pl.when(last)
    def _():
        drain_scatter(1)

    plsc.subcore_barrier()

    r0 = s * WPT
    pltpu.sync_copy(acc_sh.at[pl.ds(r0, WPT)],
                    out_hbm.at[pl.ds(r0, WPT), pl.ds(c * FH, FH)])


_sc_call = pl.kernel(
    _sc_body,
    out_type=jax.ShapeDtypeStruct((3 * N, DIN), jnp.float32),
    mesh=plsc.VectorSubcoreMesh(core_axis_name="c", subcore_axis_name="s",
                                num_cores=NC, num_subcores=NS),
    compiler_params=pltpu.CompilerParams(use_tc_tiling_on_sc=False),
    scratch_types=[
        pltpu.VMEM((2, CH, FH), jnp.float32),
        pltpu.VMEM((2, 2, CH), jnp.int32),
        pltpu.VMEM((2, 3, CH), jnp.int32),
        pltpu.VMEM_SHARED((NTOT, FH), jnp.float32),
        pltpu.SemaphoreType.DMA,
        pltpu.SemaphoreType.DMA,
        pltpu.SemaphoreType.DMA,
    ],
)


def _tc_body(acc_ref, w_ref, b_ref, y_ref):
    f32 = jnp.float32
    a_row = acc_ref[0:N, :]
    a_col = acc_ref[N:2 * N, :]
    a_diag = acc_ref[2 * N:3 * N, :]
    s_all = jnp.sum(a_row, axis=0, keepdims=True)
    s_diag = jnp.sum(a_diag, axis=0, keepdims=True)
    const = (jnp.dot(s_diag, w_ref[1], preferred_element_type=f32)
             + jnp.dot(s_all, w_ref[4], preferred_element_type=f32)
             + b_ref[0, 0])
    y = (jnp.dot(a_diag, w_ref[0], preferred_element_type=f32)
         + jnp.dot(a_row, w_ref[2], preferred_element_type=f32)
         + jnp.dot(a_col, w_ref[3], preferred_element_type=f32))
    y_ref[...] = y + const


_tc_call = pl.pallas_call(
    _tc_body,
    out_shape=jax.ShapeDtypeStruct((N, DIN), jnp.float32),
    in_specs=[
        pl.BlockSpec(memory_space=pltpu.VMEM),
        pl.BlockSpec(memory_space=pltpu.VMEM),
        pl.BlockSpec(memory_space=pltpu.SMEM),
    ],
    out_specs=pl.BlockSpec(memory_space=pltpu.VMEM),
)


@jax.jit
def kernel(values, row, col, weights, bias):
    row = row.astype(jnp.int32)
    col = col.astype(jnp.int32)
    acc = _sc_call(values, row, col)
    return _tc_call(acc, weights, bias.reshape(1, 1).astype(jnp.float32))


# speedup vs baseline: 1.0396x; 1.0216x over previous
import jax
import jax.numpy as jnp
from jax import lax
from jax.experimental import pallas as pl
from jax.experimental.pallas import tpu as pltpu
from jax.experimental.pallas import tpu_sc as plsc

N = 10000
NNZ = 320000
DIN = 128
NC = 2
NS = 16
LANES = 16
FH = DIN // NC

CH = 80
PER_TILE = NNZ // NS
NCHUNK = PER_TILE // CH

DUMP = 3 * N
NTOT = 3 * N + 1
ZPT = (3 * N) // NS
WPT = (3 * N) // NS


def _sc_body(values_hbm, row_hbm, col_hbm, out_hbm,
             vals_v, rc_v, idx_v, acc_sh,
             sem_in0, sem_in1, sem_rc0, sem_rc1, sem_sc):
    c = lax.axis_index("c")
    s = lax.axis_index("s")

    sems = (sem_in0, sem_in1)
    rsems = (sem_rc0, sem_rc1)

    def issue_inputs(k, b):
        e0 = pl.multiple_of(s * PER_TILE + k * CH, CH)
        pltpu.async_copy(values_hbm.at[pl.ds(e0, CH), pl.ds(c * FH, FH)],
                         vals_v.at[b], sems[b])
        pltpu.async_copy(row_hbm.at[pl.ds(e0, CH)], rc_v.at[b, 0], rsems[b])
        pltpu.async_copy(col_hbm.at[pl.ds(e0, CH)], rc_v.at[b, 1], rsems[b])

    def wait_rc(b):
        sem = rsems[b]
        pltpu.make_async_copy(row_hbm.at[pl.ds(0, CH)], rc_v.at[b, 0],
                              sem).wait()
        pltpu.make_async_copy(col_hbm.at[pl.ds(0, CH)], rc_v.at[b, 1],
                              sem).wait()

    def wait_vals(b):
        sem = sems[b]
        pltpu.make_async_copy(values_hbm.at[pl.ds(0, CH), pl.ds(0, FH)],
                              vals_v.at[b], sem).wait()

    def drain_scatter(n):
        for _ in range(n):
            pltpu.make_async_copy(vals_v.at[0],
                                  acc_sh.at[idx_v.at[0, 0]], sem_sc).wait()

    issue_inputs(0, 0)

    zeros = jnp.zeros((LANES,), jnp.float32)

    def zrow(i, carry):
        for q in range(FH // LANES):
            vals_v[1, i, pl.ds(q * LANES, LANES)] = zeros
        return carry

    lax.fori_loop(0, CH, zrow, 0)
    zsrc = vals_v.at[1]
    for z in range(ZPT // CH):
        pltpu.sync_copy(zsrc, acc_sh.at[pl.ds(s * ZPT + z * CH, CH)])
    if ZPT % CH:
        pltpu.sync_copy(zsrc.at[pl.ds(0, ZPT % CH)],
                        acc_sh.at[pl.ds(s * ZPT + (ZPT // CH) * CH, ZPT % CH)])

    @pl.when(s == 0)
    def _():
        pltpu.sync_copy(zsrc.at[pl.ds(0, 1)], acc_sh.at[pl.ds(DUMP, 1)])

    plsc.subcore_barrier()

    def chunk(k, b, first, last_chunk, carry, guard_issue=False):
        if not first:
            drain_scatter(2)

            @pl.when(carry)
            def _():
                drain_scatter(1)

        if not last_chunk:
            if guard_issue:
                @pl.when(k + 1 < NCHUNK)
                def _():
                    issue_inputs(k + 1, 1 - b)
            else:
                issue_inputs(k + 1, 1 - b)

        wait_rc(b)
        nd = jnp.zeros((LANES,), jnp.float32)
        for q in range(CH // LANES):
            o = q * LANES
            r = rc_v[b, 0, pl.ds(o, LANES)]
            cc = rc_v[b, 1, pl.ds(o, LANES)]
            m = r == cc
            nd = nd + jnp.where(m, 1.0, 0.0)
            idx_v[b, 0, pl.ds(o, LANES)] = r
            idx_v[b, 1, pl.ds(o, LANES)] = cc + N
            idx_v[b, 2, pl.ds(o, LANES)] = jnp.where(m, r + 2 * N, DUMP)
        tot = nd[0]
        for i in range(1, LANES):
            tot = tot + nd[i]
        ndiag = tot > 0.5

        wait_vals(b)
        vb = vals_v.at[b]
        pltpu.async_copy(vb, acc_sh.at[idx_v.at[b, 0]], sem_sc, add=True)
        pltpu.async_copy(vb, acc_sh.at[idx_v.at[b, 1]], sem_sc, add=True)

        @pl.when(ndiag)
        def _():
            pltpu.async_copy(vb, acc_sh.at[idx_v.at[b, 2]], sem_sc, add=True)

        return ndiag

    def chunk_pair(k2, carry):
        k = k2 * 2
        carry = chunk(k, 0, False, False, carry)
        carry = chunk(k + 1, 1, False, False, carry, guard_issue=True)
        return carry

    carry = chunk(0, 0, True, False, jnp.bool_(False))
    carry = chunk(1, 1, False, False, carry)
    carry = lax.fori_loop(1, NCHUNK // 2, chunk_pair, carry)
    last = carry
    drain_scatter(2)

    @pl.when(last)
    def _():
        drain_scatter(1)

    plsc.subcore_barrier()

    r0 = s * WPT
    pltpu.sync_copy(acc_sh.at[pl.ds(r0, WPT)],
                    out_hbm.at[pl.ds(r0, WPT), pl.ds(c * FH, FH)])


_sc_call = pl.kernel(
    _sc_body,
    out_type=jax.ShapeDtypeStruct((3 * N, DIN), jnp.float32),
    mesh=plsc.VectorSubcoreMesh(core_axis_name="c", subcore_axis_name="s",
                                num_cores=NC, num_subcores=NS),
    compiler_params=pltpu.CompilerParams(use_tc_tiling_on_sc=False),
    scratch_types=[
        pltpu.VMEM((2, CH, FH), jnp.float32),
        pltpu.VMEM((2, 2, CH), jnp.int32),
        pltpu.VMEM((2, 3, CH), jnp.int32),
        pltpu.VMEM_SHARED((NTOT, FH), jnp.float32),
        pltpu.SemaphoreType.DMA,
        pltpu.SemaphoreType.DMA,
        pltpu.SemaphoreType.DMA,
        pltpu.SemaphoreType.DMA,
        pltpu.SemaphoreType.DMA,
    ],
)


def _tc_body(acc_ref, w_ref, b_ref, y_ref):
    f32 = jnp.float32
    a_row = acc_ref[0:N, :]
    a_col = acc_ref[N:2 * N, :]
    a_diag = acc_ref[2 * N:3 * N, :]
    s_all = jnp.sum(a_row, axis=0, keepdims=True)
    s_diag = jnp.sum(a_diag, axis=0, keepdims=True)
    const = (jnp.dot(s_diag, w_ref[1], preferred_element_type=f32)
             + jnp.dot(s_all, w_ref[4], preferred_element_type=f32)
             + b_ref[0, 0])
    y = (jnp.dot(a_diag, w_ref[0], preferred_element_type=f32)
         + jnp.dot(a_row, w_ref[2], preferred_element_type=f32)
         + jnp.dot(a_col, w_ref[3], preferred_element_type=f32))
    y_ref[...] = y + const


_tc_call = pl.pallas_call(
    _tc_body,
    out_shape=jax.ShapeDtypeStruct((N, DIN), jnp.float32),
    in_specs=[
        pl.BlockSpec(memory_space=pltpu.VMEM),
        pl.BlockSpec(memory_space=pltpu.VMEM),
        pl.BlockSpec(memory_space=pltpu.SMEM),
    ],
    out_specs=pl.BlockSpec(memory_space=pltpu.VMEM),
)


@jax.jit
def kernel(values, row, col, weights, bias):
    row = row.astype(jnp.int32)
    col = col.astype(jnp.int32)
    acc = _sc_call(values, row, col)
    return _tc_call(acc, weights, bias.reshape(1, 1).astype(jnp.float32))
